# baseline (speedup 1.0000x reference)
import jax
import jax.numpy as jnp
from jax.experimental import pallas as pl
from jax.experimental.pallas import tpu as pltpu

_CB = 128
_NBUF = 4
_OC = 1024


def _fused_body(w1_ref, b1_ref, w2_ref, b2_ref, x_hbm, a_hbm, out_hbm,
                ahat, stage, x_vmem, out_stage, sem, x_sem, out_sem):
    n = ahat.shape[0]
    d = x_vmem.shape[1]
    nchunks = n // _CB

    copies = [
        pltpu.make_async_copy(a_hbm.at[pl.ds(k * _CB, _CB), :],
                              stage.at[k % _NBUF], sem.at[k % _NBUF])
        for k in range(nchunks)
    ]
    x_copy = pltpu.make_async_copy(x_hbm, x_vmem, x_sem)
    for k in range(_NBUF - 1):
        copies[k].start()
    x_copy.start()
    ones = jnp.ones((1, _CB), jnp.bfloat16)
    deg = jnp.zeros((1, n), jnp.float32)
    hw1 = None
    for k in range(nchunks):
        if k + _NBUF - 1 < nchunks:
            copies[k + _NBUF - 1].start()
        copies[k].wait()
        a = stage[k % _NBUF]
        rows = jax.lax.broadcasted_iota(jnp.int32, (_CB, n), 0) + k * _CB
        cols = jax.lax.broadcasted_iota(jnp.int32, (_CB, n), 1)
        ablk = jnp.where(rows == cols, 1, a).astype(jnp.bfloat16)
        ahat[pl.ds(k * _CB, _CB), :] = ablk
        deg += jax.lax.dot_general(
            ones, ablk, (((1,), (0,)), ((), ())),
            preferred_element_type=jnp.float32)
        if k == nchunks // 2:
            x_copy.wait()
            hw1 = jax.lax.dot_general(
                w1_ref[...], x_vmem[...], (((0,), (1,)), ((), ())),
                preferred_element_type=jnp.float32)

    dinv = jnp.where(deg > 0, jax.lax.rsqrt(deg), 0.0)

    g1 = (dinv * hw1).astype(jnp.bfloat16)
    b1_col = b1_ref[...].reshape(d, 1)
    b2_col = b2_ref[...].reshape(d, 1)
    g2_chunks = []
    for c in range(n // _OC):
        sl = slice(c * _OC, (c + 1) * _OC)
        acc1 = jax.lax.dot_general(
            g1, ahat[:, sl], (((1,), (0,)), ((), ())),
            preferred_element_type=jnp.float32)
        h1c = jnp.maximum(acc1 * dinv[:, sl] + b1_col, 0.0)
        hw2c = jax.lax.dot_general(
            w2_ref[...], h1c, (((0,), (0,)), ((), ())),
            preferred_element_type=jnp.float32)
        g2_chunks.append((dinv[:, sl] * hw2c).astype(jnp.bfloat16))
    g2 = jnp.concatenate(g2_chunks, axis=1)

    nout = n // _OC
    out_copies = [
        pltpu.make_async_copy(out_stage.at[c % 2],
                              out_hbm.at[pl.ds(c * _OC, _OC), :],
                              out_sem.at[c % 2])
        for c in range(nout)
    ]
    for c in range(nout):
        acc2 = jax.lax.dot_general(
            g2, ahat[:, c * _OC:(c + 1) * _OC], (((1,), (0,)), ((), ())),
            preferred_element_type=jnp.float32)
        res = jnp.maximum(acc2 * dinv[:, c * _OC:(c + 1) * _OC] + b2_col, 0.0)
        if c >= 2:
            out_copies[c - 2].wait()
        out_stage[c % 2] = res.T
        out_copies[c].start()
    for c in range(max(nout - 2, 0), nout):
        out_copies[c].wait()


@jax.jit
def _gcn_block(x, adj_matrix, W1, b1, W2, b2):
    n, d = x.shape
    return pl.pallas_call(
        _fused_body,
        in_specs=[
            pl.BlockSpec(memory_space=pltpu.VMEM),
            pl.BlockSpec(memory_space=pltpu.VMEM),
            pl.BlockSpec(memory_space=pltpu.VMEM),
            pl.BlockSpec(memory_space=pltpu.VMEM),
            pl.BlockSpec(memory_space=pl.ANY),
            pl.BlockSpec(memory_space=pl.ANY),
        ],
        out_specs=pl.BlockSpec(memory_space=pl.ANY),
        out_shape=jax.ShapeDtypeStruct((n, d), jnp.float32),
        scratch_shapes=[
            pltpu.VMEM((n, n), jnp.bfloat16),
            pltpu.VMEM((_NBUF, _CB, n), jnp.int32),
            pltpu.VMEM((n, d), jnp.float32),
            pltpu.VMEM((2, _OC, d), jnp.float32),
            pltpu.SemaphoreType.DMA((_NBUF,)),
            pltpu.SemaphoreType.DMA,
            pltpu.SemaphoreType.DMA((2,)),
        ],
    )(W1, b1.reshape(1, d), W2, b2.reshape(1, d), x, adj_matrix)


def kernel(x, adj_matrix, W1, b1, W2, b2):
    return _gcn_block(x, adj_matrix, W1, b1, W2, b2)


# speedup vs baseline: 1.0054x; 1.0054x over previous
import jax
import jax.numpy as jnp
from jax.experimental import pallas as pl
from jax.experimental.pallas import tpu as pltpu

_CB = 128
_NBUF = 4
_OC = 1024


def _fused_body(w1_ref, b1_ref, w2_ref, b2_ref, x_hbm, a_hbm, out_hbm,
                ahat, stage, x_vmem, out_stage, g2_buf, sem, x_sem, out_sem):
    n = ahat.shape[0]
    d = x_vmem.shape[1]
    nchunks = n // _CB

    copies = [
        pltpu.make_async_copy(a_hbm.at[pl.ds(k * _CB, _CB), :],
                              stage.at[k % _NBUF], sem.at[k % _NBUF])
        for k in range(nchunks)
    ]
    x_copy = pltpu.make_async_copy(x_hbm, x_vmem, x_sem)
    for k in range(_NBUF - 1):
        copies[k].start()
    x_copy.start()
    ones = jnp.ones((1, _CB), jnp.bfloat16)
    deg = jnp.zeros((1, n), jnp.float32)
    hw1 = None
    for k in range(nchunks):
        if k + _NBUF - 1 < nchunks:
            copies[k + _NBUF - 1].start()
        copies[k].wait()
        a = stage[k % _NBUF]
        rows = jax.lax.broadcasted_iota(jnp.int32, (_CB, n), 0) + k * _CB
        cols = jax.lax.broadcasted_iota(jnp.int32, (_CB, n), 1)
        ablk = jnp.where(rows == cols, 1, a).astype(jnp.bfloat16)
        ahat[pl.ds(k * _CB, _CB), :] = ablk
        deg += jax.lax.dot_general(
            ones, ablk, (((1,), (0,)), ((), ())),
            preferred_element_type=jnp.float32)
        if k == nchunks // 2:
            x_copy.wait()
            hw1 = jax.lax.dot_general(
                w1_ref[...], x_vmem[...], (((0,), (1,)), ((), ())),
                preferred_element_type=jnp.float32)

    dinv = jnp.where(deg > 0, jax.lax.rsqrt(deg), 0.0)

    g1 = (dinv * hw1).astype(jnp.bfloat16)
    b1_col = b1_ref[...].reshape(d, 1)
    b2_col = b2_ref[...].reshape(d, 1)
    for c in range(n // _OC):
        sl = slice(c * _OC, (c + 1) * _OC)
        acc1 = jax.lax.dot_general(
            g1, ahat[:, sl], (((1,), (0,)), ((), ())),
            preferred_element_type=jnp.float32)
        h1c = jnp.maximum(acc1 * dinv[:, sl] + b1_col, 0.0)
        hw2c = jax.lax.dot_general(
            w2_ref[...], h1c, (((0,), (0,)), ((), ())),
            preferred_element_type=jnp.float32)
        g2_buf[:, sl] = (dinv[:, sl] * hw2c).astype(jnp.bfloat16)
    g2 = g2_buf[...]

    nout = n // _OC
    out_copies = [
        pltpu.make_async_copy(out_stage.at[c % 2],
                              out_hbm.at[pl.ds(c * _OC, _OC), :],
                              out_sem.at[c % 2])
        for c in range(nout)
    ]
    for c in range(nout):
        acc2 = jax.lax.dot_general(
            g2, ahat[:, c * _OC:(c + 1) * _OC], (((1,), (0,)), ((), ())),
            preferred_element_type=jnp.float32)
        res = jnp.maximum(acc2 * dinv[:, c * _OC:(c + 1) * _OC] + b2_col, 0.0)
        if c >= 2:
            out_copies[c - 2].wait()
        out_stage[c % 2] = res.T
        out_copies[c].start()
    for c in range(max(nout - 2, 0), nout):
        out_copies[c].wait()


@jax.jit
def _gcn_block(x, adj_matrix, W1, b1, W2, b2):
    n, d = x.shape
    return pl.pallas_call(
        _fused_body,
        in_specs=[
            pl.BlockSpec(memory_space=pltpu.VMEM),
            pl.BlockSpec(memory_space=pltpu.VMEM),
            pl.BlockSpec(memory_space=pltpu.VMEM),
            pl.BlockSpec(memory_space=pltpu.VMEM),
            pl.BlockSpec(memory_space=pl.ANY),
            pl.BlockSpec(memory_space=pl.ANY),
        ],
        out_specs=pl.BlockSpec(memory_space=pl.ANY),
        out_shape=jax.ShapeDtypeStruct((n, d), jnp.float32),
        scratch_shapes=[
            pltpu.VMEM((n, n), jnp.bfloat16),
            pltpu.VMEM((_NBUF, _CB, n), jnp.int32),
            pltpu.VMEM((n, d), jnp.float32),
            pltpu.VMEM((2, _OC, d), jnp.float32),
            pltpu.VMEM((d, n), jnp.bfloat16),
            pltpu.SemaphoreType.DMA((_NBUF,)),
            pltpu.SemaphoreType.DMA,
            pltpu.SemaphoreType.DMA((2,)),
        ],
    )(W1, b1.reshape(1, d), W2, b2.reshape(1, d), x, adj_matrix)


def kernel(x, adj_matrix, W1, b1, W2, b2):
    return _gcn_block(x, adj_matrix, W1, b1, W2, b2)


# speedup vs baseline: 1.0226x; 1.0171x over previous
import jax
import jax.numpy as jnp
from jax.experimental import pallas as pl
from jax.experimental.pallas import tpu as pltpu

_CB = 128
_NBUF = 4
_OC = 1024


def _fused_body(w1_ref, b1_ref, w2_ref, b2_ref, x_hbm, a_hbm, out_hbm,
                ahat, stage, x_vmem, out_stage, sem, x_sem, out_sem):
    n = ahat.shape[0]
    d = x_vmem.shape[1]
    nchunks = n // _CB

    copies = [
        pltpu.make_async_copy(a_hbm.at[pl.ds(k * _CB, _CB), :],
                              stage.at[k % _NBUF], sem.at[k % _NBUF])
        for k in range(nchunks)
    ]
    x_copy = pltpu.make_async_copy(x_hbm, x_vmem, x_sem)
    for k in range(_NBUF - 1):
        copies[k].start()
    x_copy.start()
    ones = jnp.ones((1, _CB), jnp.bfloat16)
    deg = jnp.zeros((1, n), jnp.float32)
    hw1 = None
    for k in range(nchunks):
        if k + _NBUF - 1 < nchunks:
            copies[k + _NBUF - 1].start()
        copies[k].wait()
        a = stage[k % _NBUF]
        rows = jax.lax.broadcasted_iota(jnp.int32, (_CB, n), 0) + k * _CB
        cols = jax.lax.broadcasted_iota(jnp.int32, (_CB, n), 1)
        ablk = jnp.where(rows == cols, 1, a).astype(jnp.bfloat16)
        ahat[pl.ds(k * _CB, _CB), :] = ablk
        deg += jax.lax.dot_general(
            ones, ablk, (((1,), (0,)), ((), ())),
            preferred_element_type=jnp.float32)
        if k == nchunks // 2:
            x_copy.wait()
            hw1 = jax.lax.dot_general(
                w1_ref[...], x_vmem[...], (((0,), (1,)), ((), ())),
                preferred_element_type=jnp.float32)

    dinv = jnp.where(deg > 0, jax.lax.rsqrt(deg), 0.0)

    g1 = (dinv * hw1).astype(jnp.bfloat16)
    acc1 = jax.lax.dot_general(
        g1, ahat[...], (((1,), (0,)), ((), ())),
        preferred_element_type=jnp.float32)
    h1 = jnp.maximum(acc1 * dinv + b1_ref[...].reshape(d, 1), 0.0)
    hw2 = jax.lax.dot_general(
        w2_ref[...], h1, (((0,), (0,)), ((), ())),
        preferred_element_type=jnp.float32)
    g2 = (dinv * hw2).astype(jnp.bfloat16)
    b2_col = b2_ref[...].reshape(d, 1)

    nout = n // _OC
    out_copies = [
        pltpu.make_async_copy(out_stage.at[c % 2],
                              out_hbm.at[pl.ds(c * _OC, _OC), :],
                              out_sem.at[c % 2])
        for c in range(nout)
    ]
    for c in range(nout):
        acc2 = jax.lax.dot_general(
            g2, ahat[:, c * _OC:(c + 1) * _OC], (((1,), (0,)), ((), ())),
            preferred_element_type=jnp.float32)
        res = jnp.maximum(acc2 * dinv[:, c * _OC:(c + 1) * _OC] + b2_col, 0.0)
        if c >= 2:
            out_copies[c - 2].wait()
        out_stage[c % 2] = res.T
        out_copies[c].start()
    for c in range(max(nout - 2, 0), nout):
        out_copies[c].wait()


@jax.jit
def _gcn_block(x, adj_matrix, W1, b1, W2, b2):
    n, d = x.shape
    return pl.pallas_call(
        _fused_body,
        in_specs=[
            pl.BlockSpec(memory_space=pltpu.VMEM),
            pl.BlockSpec(memory_space=pltpu.VMEM),
            pl.BlockSpec(memory_space=pltpu.VMEM),
            pl.BlockSpec(memory_space=pltpu.VMEM),
            pl.BlockSpec(memory_space=pl.ANY),
            pl.BlockSpec(memory_space=pl.ANY),
        ],
        out_specs=pl.BlockSpec(memory_space=pl.ANY),
        out_shape=jax.ShapeDtypeStruct((n, d), jnp.float32),
        scratch_shapes=[
            pltpu.VMEM((n, n), jnp.bfloat16),
            pltpu.VMEM((_NBUF, _CB, n), jnp.int32),
            pltpu.VMEM((n, d), jnp.float32),
            pltpu.VMEM((2, _OC, d), jnp.float32),
            pltpu.SemaphoreType.DMA((_NBUF,)),
            pltpu.SemaphoreType.DMA,
            pltpu.SemaphoreType.DMA((2,)),
        ],
    )(W1, b1.reshape(1, d), W2, b2.reshape(1, d), x, adj_matrix)


def kernel(x, adj_matrix, W1, b1, W2, b2):
    return _gcn_block(x, adj_matrix, W1, b1, W2, b2)
